# Initial kernel scaffold; baseline (speedup 1.0000x reference)
import jax
import jax.numpy as jnp
from jax.experimental import pallas as pl


def kernel(current_entity, source_entity, query_relation, encoded_history, r_space, e_space, action_mask, entity_emb, relation_emb, W1, b1, W2, b2):
    raise NotImplementedError("write your pallas kernel here")


# speedup vs baseline: 10.6466x; 10.6466x over previous
import functools

import jax
import jax.numpy as jnp
from jax import lax
from jax.experimental import pallas as pl
from jax.experimental.pallas import tpu as pltpu
from jax.experimental.pallas import tpu_sc as plsc

NUM_ENTITIES = 100000
NUM_RELATIONS = 474
D = 200
B = 2048
A = 256
RPAD = 480
HUGE = 1e31

NC, NS = 2, 16
NW = NC * NS
ROWS_W = B // NW
CHUNK = 128

_sc_mesh = functools.partial(
    plsc.VectorSubcoreMesh, core_axis_name="c", subcore_axis_name="s",
    num_cores=NC, num_subcores=NS)

_SC_PARAMS = pltpu.CompilerParams(use_tc_tiling_on_sc=False,
                                  needs_layout_passes=False)


def _wid():
    return lax.axis_index("s") * NC + lax.axis_index("c")


def _gather_eq_body(ent_hbm, rel_hbm, ce_hbm, qr_hbm, e_out, q_out,
                    eidx, qidx, erows, qrows, sem_e, sem_q):
    base = _wid() * ROWS_W
    pltpu.sync_copy(ce_hbm.at[pl.ds(base, ROWS_W)], eidx)
    pltpu.sync_copy(qr_hbm.at[pl.ds(base, ROWS_W)], qidx)
    cp_e = pltpu.async_copy(ent_hbm.at[eidx], erows, sem_e)
    cp_q = pltpu.async_copy(rel_hbm.at[qidx], qrows, sem_q)
    cp_e.wait()
    cp_q.wait()
    pltpu.sync_copy(erows, e_out.at[pl.ds(base, ROWS_W)])
    pltpu.sync_copy(qrows, q_out.at[pl.ds(base, ROWS_W)])


@functools.cache
def _gather_eq():
    return pl.kernel(
        _gather_eq_body,
        out_type=[jax.ShapeDtypeStruct((B, D), jnp.float32),
                  jax.ShapeDtypeStruct((B, D), jnp.float32)],
        mesh=_sc_mesh(),
        scratch_types=[
            pltpu.VMEM((ROWS_W,), jnp.int32),
            pltpu.VMEM((ROWS_W,), jnp.int32),
            pltpu.VMEM((ROWS_W, D), jnp.float32),
            pltpu.VMEM((ROWS_W, D), jnp.float32),
            pltpu.SemaphoreType.DMA,
            pltpu.SemaphoreType.DMA,
        ],
        compiler_params=_SC_PARAMS,
    )


def _policy_body(e_ref, h_ref, q_ref, w1a_ref, w1b_ref, w1c_ref, b1_ref,
                 w2a_ref, w2e_ref, b2a_ref, b2e_ref, relt_ref,
                 x2e_ref, rl_ref):
    f32 = jnp.float32
    h1 = (jnp.dot(e_ref[...], w1a_ref[...], preferred_element_type=f32)
          + jnp.dot(h_ref[...], w1b_ref[...], preferred_element_type=f32)
          + jnp.dot(q_ref[...], w1c_ref[...], preferred_element_type=f32)
          + b1_ref[...])
    h1 = jnp.maximum(h1, 0.0)
    x2a = jnp.dot(h1, w2a_ref[...], preferred_element_type=f32) + b2a_ref[...]
    x2e = jnp.dot(h1, w2e_ref[...], preferred_element_type=f32) + b2e_ref[...]
    x2e_ref[...] = x2e
    rl_ref[...] = jnp.dot(x2a, relt_ref[...], preferred_element_type=f32)


_BB = 256


def _policy(e, h, q, w1a, w1b, w1c, b1, w2a, w2e, b2a, b2e, relt):
    nblk = B // _BB
    row_spec = lambda cols: pl.BlockSpec((_BB, cols), lambda i: (i, 0))
    full = lambda shape: pl.BlockSpec(shape, lambda i: tuple(0 for _ in shape))
    return pl.pallas_call(
        _policy_body,
        grid=(nblk,),
        in_specs=[row_spec(D), row_spec(D), row_spec(D),
                  full((D, 2 * D)), full((D, 2 * D)), full((D, 2 * D)),
                  full((1, 2 * D)),
                  full((2 * D, D)), full((2 * D, D)),
                  full((1, D)), full((1, D)),
                  full((D, RPAD))],
        out_specs=[row_spec(D), row_spec(RPAD)],
        out_shape=[jax.ShapeDtypeStruct((B, D), jnp.float32),
                   jax.ShapeDtypeStruct((B, RPAD), jnp.float32)],
    )(e, h, q, w1a, w1b, w1c, b1, w2a, w2e, b2a, b2e, relt)


def _logits_sc_body(ent_hbm, es_hbm, rs_hbm, rl_hbm, x2_hbm, out_hbm,
                    es_v, rs_v, rl_v, x_v, rows_v, lg_v, sem0, sem1):
    base = _wid() * ROWS_W
    pltpu.sync_copy(rl_hbm.at[pl.ds(base, ROWS_W)], rl_v)
    pltpu.sync_copy(x2_hbm.at[pl.ds(base, ROWS_W)], x_v)
    lane = lax.iota(jnp.int32, 16)

    def row_body(i, carry):
        b = base + i
        pltpu.sync_copy(es_hbm.at[b], es_v)
        cp0 = pltpu.async_copy(ent_hbm.at[es_v.at[0]], rows_v.at[0], sem0)
        cp1 = pltpu.async_copy(ent_hbm.at[es_v.at[1]], rows_v.at[1], sem1)
        pltpu.sync_copy(rs_hbm.at[b], rs_v)
        cp0.wait()
        cp1.wait()
        ivec = jnp.full((16,), i, jnp.int32)
        zero = jnp.zeros((16,), jnp.float32)
        for c in range(2):
            rows_c = rows_v.at[c]

            def blk_body(blk, carry2, rows_c=rows_c, c=c):
                aidx = lane + blk * 16

                def dstep(j, accs, aidx=aidx, rows_c=rows_c):
                    d0 = 16 * j
                    xc = x_v[i, pl.ds(d0, 16)]
                    accs = list(accs)
                    for k in range(16):
                        v = plsc.load_gather(
                            rows_c, [aidx, jnp.full((16,), d0 + k, jnp.int32)])
                        accs[k % 8] = accs[k % 8] + v * xc[k]
                    return tuple(accs)

                accs = lax.fori_loop(0, 12, dstep, (zero,) * 8)
                xc = x_v[i, pl.ds(184, 16)]
                accs = list(accs)
                for k in range(8, 16):
                    v = plsc.load_gather(
                        rows_c, [aidx, jnp.full((16,), 184 + k, jnp.int32)])
                    accs[k % 8] = accs[k % 8] + v * xc[k]
                acc = (((accs[0] + accs[1]) + (accs[2] + accs[3]))
                       + ((accs[4] + accs[5]) + (accs[6] + accs[7])))
                ridx = rs_v[pl.ds(c * CHUNK + blk * 16, 16)]
                rvals = plsc.load_gather(rl_v, [ivec, ridx])
                lg_v[pl.ds(c * CHUNK + blk * 16, 16)] = acc + rvals
                return carry2

            lax.fori_loop(0, CHUNK // 16, blk_body, 0)
        pltpu.sync_copy(lg_v, out_hbm.at[b])
        return carry

    lax.fori_loop(0, ROWS_W, row_body, 0)


@functools.cache
def _logits_sc():
    return pl.kernel(
        _logits_sc_body,
        out_type=jax.ShapeDtypeStruct((B, A), jnp.float32),
        mesh=_sc_mesh(),
        scratch_types=[
            pltpu.VMEM((2, CHUNK), jnp.int32),
            pltpu.VMEM((A,), jnp.int32),
            pltpu.VMEM((ROWS_W, RPAD), jnp.float32),
            pltpu.VMEM((ROWS_W, D), jnp.float32),
            pltpu.VMEM((2, CHUNK, D), jnp.float32),
            pltpu.VMEM((A,), jnp.float32),
            pltpu.SemaphoreType.DMA,
            pltpu.SemaphoreType.DMA,
        ],
        compiler_params=_SC_PARAMS,
    )


def _softmax_body(lg_ref, mask_ref, p_ref, ent_ref):
    l = lg_ref[...] - (1.0 - mask_ref[...]) * HUGE
    m = jnp.max(l, axis=1, keepdims=True)
    e = jnp.exp(l - m)
    s = jnp.sum(e, axis=1, keepdims=True)
    p = e / s
    p_ref[...] = p
    ent_ref[...] = -jnp.sum(p * jnp.log(p + 1e-20), axis=1, keepdims=True)


def _softmax_entropy(logits, mask):
    nblk = B // _BB
    spec = pl.BlockSpec((_BB, A), lambda i: (i, 0))
    return pl.pallas_call(
        _softmax_body,
        grid=(nblk,),
        in_specs=[spec, spec],
        out_specs=[spec, pl.BlockSpec((_BB, 1), lambda i: (i, 0))],
        out_shape=[jax.ShapeDtypeStruct((B, A), jnp.float32),
                   jax.ShapeDtypeStruct((B, 1), jnp.float32)],
    )(logits, mask)


def kernel(current_entity, source_entity, query_relation, encoded_history,
           r_space, e_space, action_mask, entity_emb, relation_emb,
           W1, b1, W2, b2):
    f32 = jnp.float32
    ce = current_entity.astype(jnp.int32)
    qr = query_relation.astype(jnp.int32)
    es3 = e_space.astype(jnp.int32).reshape(B, 2, CHUNK)
    rs = r_space.astype(jnp.int32)

    E, Q = _gather_eq()(entity_emb, relation_emb, ce, qr)

    relt = jnp.zeros((D, RPAD), f32).at[:, :NUM_RELATIONS].set(relation_emb.T)
    X2e, RL = _policy(
        E, encoded_history, Q,
        W1[:D], W1[D:2 * D], W1[2 * D:], b1.reshape(1, -1),
        W2[:, :D], W2[:, D:], b2[:D].reshape(1, -1), b2[D:].reshape(1, -1),
        relt)

    logits = _logits_sc()(entity_emb, es3, rs, RL, X2e)

    p, ent = _softmax_entropy(logits, action_mask)
    return (p, ent.reshape(-1))
